# baseline (speedup 1.0000x reference)
import functools

import jax
import jax.numpy as jnp
from jax import lax
from jax.experimental import pallas as pl
from jax.experimental.pallas import tpu as pltpu
from jax.experimental.pallas import tpu_sc as plsc

K = 8
B, D, N = 4, 2048, 8192
R = B * D
L = 16
VPR = N // L
G = 8
NG = VPR // G
NC, NS = 2, 16
NW = NC * NS
RPW = R // NW
CR = 4
CW = CR * N
NCH = RPW // CR
NEG = float("-inf")


def _sortd(v):
    sk, _ = plsc.sort_key_val(v, v, descending=True)
    return sk


def _merge8(t8, v, lane):
    sv = _sortd(v)
    rv = lax.rev(sv, (0,))
    comb = jnp.where(lane < K, t8, rv)
    return _sortd(comb)


def _sc_body(x_hbm, out_hbm, buf, gmax, stage, sem0, sem1):
    cid = lax.axis_index("c")
    sid = lax.axis_index("s")
    w = sid * NC + cid
    row0 = w * RPW
    base_off = row0 * N
    b_idx = w // (D // RPW)
    d0 = (w % (D // RPW)) * RPW

    lane = lax.iota(jnp.int32, L)
    lt8 = lane < K
    sems = (sem0, sem1)

    def start_chunk(c, par):
        pltpu.make_async_copy(
            x_hbm.at[pl.ds(base_off + c * CW, CW)],
            buf.at[pl.ds(par * CW, CW)],
            sems[par],
        ).start()

    for par in range(2):
        start_chunk(par, par)

    def chunk_pair(s, carry):
        for par in range(2):
            c = 2 * s + par
            pltpu.make_async_copy(
                x_hbm.at[pl.ds(base_off + c * CW, CW)],
                buf.at[pl.ds(par * CW, CW)],
                sems[par],
            ).wait()
            pbase = par * CW

            def row_body(r, _):
                rb = pbase + r * N

                def g1(g, m_run):
                    gb = rb + g * (G * L)
                    gm = buf[pl.ds(gb, L)]
                    for i in range(1, G):
                        gm = jnp.maximum(gm, buf[pl.ds(gb + i * L, L)])
                    gmax[pl.ds(g * L, L)] = gm
                    return jnp.maximum(m_run, gm)

                m_run = lax.fori_loop(
                    0, NG, g1, jnp.full((L,), NEG, jnp.float32))
                sm = _sortd(m_run)
                t_scalar = jnp.max(jnp.where(lane == K - 1, sm, NEG))
                t_vec = jnp.full((L,), t_scalar, jnp.float32)

                def g2(g, top8):
                    gm = gmax[pl.ds(g * L, L)]
                    anyq = jnp.any(gm >= t_vec)

                    def scan_group(t8):
                        gb = rb + g * (G * L)
                        for i in range(G):
                            x = buf[pl.ds(gb + i * L, L)]
                            mask = x >= t_vec
                            anyv = jnp.any(mask)
                            t8 = lax.cond(
                                anyv,
                                lambda tt: _merge8(
                                    tt, jnp.where(mask, x, NEG), lane),
                                lambda tt: tt,
                                t8,
                            )
                        return t8

                    return lax.cond(anyq, scan_group, lambda tt: tt, top8)

                top8 = lax.fori_loop(
                    0, NG, g2, jnp.full((L,), NEG, jnp.float32))

                i_row = c * CR + r
                plsc.store_scatter(
                    stage, [lane * RPW + i_row], top8, mask=lt8)
                return 0

            lax.fori_loop(0, CR, row_body, 0)

            @pl.when(c + 2 < NCH)
            def _():
                start_chunk(c + 2, par)

        return carry

    lax.fori_loop(0, NCH // 2, chunk_pair, 0)

    for j in range(K):
        pltpu.sync_copy(
            stage.at[pl.ds(j * RPW, RPW)],
            out_hbm.at[b_idx, j, pl.ds(d0, RPW)],
        )


@functools.partial(
    pl.kernel,
    out_type=jax.ShapeDtypeStruct((B, K, D), jnp.float32),
    mesh=plsc.VectorSubcoreMesh(core_axis_name="c", subcore_axis_name="s"),
    compiler_params=pltpu.CompilerParams(needs_layout_passes=False),
    scratch_types=[
        pltpu.VMEM((2 * CW,), jnp.float32),
        pltpu.VMEM((NG * L,), jnp.float32),
        pltpu.VMEM((K * RPW,), jnp.float32),
        pltpu.SemaphoreType.DMA,
        pltpu.SemaphoreType.DMA,
    ],
)
def _sc_topk(x_hbm, out_hbm, buf, gmax, stage, sem0, sem1):
    _sc_body(x_hbm, out_hbm, buf, gmax, stage, sem0, sem1)


def kernel(inputs):
    return _sc_topk(inputs.reshape(-1))
